# Initial kernel scaffold; baseline (speedup 1.0000x reference)
import functools

import jax
import jax.numpy as jnp
from jax import lax
from jax.experimental import pallas as pl
from jax.experimental.pallas import tpu as pltpu
from jax.experimental.pallas import tpu_sc as plsc

N = 10000
E = 320000
D = 128
G = 256

NC = 2
NS = 16
NW = NC * NS

NPAD = 10240
GPAD = 288
TRASH_N = NPAD - 1
TRASH_G = GPAD - 1

EPT = E // NW
ECH = 79
PCH = 3

NPT = NPAD // NS
GPT = GPAD // NS


def _tile_id():
    return lax.axis_index("c") * NS + lax.axis_index("s")


def _counts_body(dst_idx, pool_idx, ones16, z16,
                 deg_out, cnt_out,
                 idx_v, pidx_v, ones_v, deg_sh, cnt_sh):
    c = lax.axis_index("c")
    s = lax.axis_index("s")
    w = _tile_id()
    pltpu.sync_copy(z16.at[pl.ds(s * NPT, NPT)], deg_sh.at[pl.ds(s * NPT, NPT)])
    pltpu.sync_copy(z16.at[pl.ds(s * GPT, GPT)], cnt_sh.at[pl.ds(s * GPT, GPT)])
    pltpu.sync_copy(ones16, ones_v)
    pltpu.sync_copy(dst_idx.at[w], idx_v)
    pltpu.sync_copy(pool_idx.at[w], pidx_v)
    plsc.subcore_barrier()

    def chunk(i, carry):
        pltpu.sync_copy(ones_v, deg_sh.at[idx_v.at[i]], add=True)
        return carry

    lax.fori_loop(0, ECH, chunk, 0)
    for j in range(PCH):
        pltpu.sync_copy(ones_v, cnt_sh.at[pidx_v.at[j]], add=True)
    plsc.subcore_barrier()
    pltpu.sync_copy(deg_sh.at[pl.ds(s * NPT, NPT)],
                    deg_out.at[c, pl.ds(s * NPT, NPT)])
    pltpu.sync_copy(cnt_sh.at[pl.ds(s * GPT, GPT)],
                    cnt_out.at[c, pl.ds(s * GPT, GPT)])


def _sc_counts(dst_idx, pool_idx, ones16, z16):
    mesh = plsc.VectorSubcoreMesh(core_axis_name="c", subcore_axis_name="s")
    fn = pl.kernel(
        _counts_body,
        out_type=(jax.ShapeDtypeStruct((NC, NPAD, 16), jnp.float32),
                  jax.ShapeDtypeStruct((NC, GPAD, 16), jnp.float32)),
        mesh=mesh,
        scratch_types=[
            pltpu.VMEM((ECH, 128), jnp.int32),
            pltpu.VMEM((PCH, 128), jnp.int32),
            pltpu.VMEM((128, 16), jnp.float32),
            pltpu.VMEM_SHARED((NPAD, 16), jnp.float32),
            pltpu.VMEM_SHARED((GPAD, 16), jnp.float32),
        ],
    )
    return fn(dst_idx, pool_idx, ones16, z16)


def _gs_body(nch, rows_pt, src_idx, dst_idx, table, zrows, agg_out,
             sidx_v, didx_v, rows_v, acc_sh, sem):
    c = lax.axis_index("c")
    s = lax.axis_index("s")
    w = _tile_id()
    pltpu.sync_copy(zrows.at[pl.ds(s * rows_pt, rows_pt)],
                    acc_sh.at[pl.ds(s * rows_pt, rows_pt)])
    pltpu.sync_copy(src_idx.at[w], sidx_v)
    pltpu.sync_copy(dst_idx.at[w], didx_v)
    plsc.subcore_barrier()

    def chunk(i, carry):
        pltpu.async_copy(table.at[sidx_v.at[i]], rows_v, sem).wait()
        pltpu.sync_copy(rows_v, acc_sh.at[didx_v.at[i]], add=True)
        return carry

    lax.fori_loop(0, nch, chunk, 0)
    plsc.subcore_barrier()
    pltpu.sync_copy(acc_sh.at[pl.ds(s * rows_pt, rows_pt)],
                    agg_out.at[c, pl.ds(s * rows_pt, rows_pt)])


def _sc_gather_scatter(nch, acc_rows, src_idx, dst_idx, table, zrows):
    rows_pt = acc_rows // NS
    mesh = plsc.VectorSubcoreMesh(core_axis_name="c", subcore_axis_name="s")
    fn = pl.kernel(
        functools.partial(_gs_body, nch, rows_pt),
        out_type=jax.ShapeDtypeStruct((NC, acc_rows, 64), jnp.float32),
        mesh=mesh,
        scratch_types=[
            pltpu.VMEM((nch, 128), jnp.int32),
            pltpu.VMEM((nch, 128), jnp.int32),
            pltpu.VMEM((128, 64), jnp.float32),
            pltpu.VMEM_SHARED((acc_rows, 64), jnp.float32),
            pltpu.SemaphoreType.DMA,
        ],
    )
    return fn(src_idx, dst_idx, table, zrows)


def _dinv_block(degb):
    deg = degb[0, :, 0] + degb[1, :, 0] + 1.0
    return lax.rsqrt(deg)[:, None]


def _tc1_body(xb, w1, degb, ob):
    h = jnp.dot(xb[...], w1[...], preferred_element_type=jnp.float32)
    ob[...] = h * _dinv_block(degb[...])


def _tc_pre(x_pad, W1, deg_part):
    return pl.pallas_call(
        _tc1_body,
        grid=(NPAD // 256,),
        in_specs=[
            pl.BlockSpec((256, D), lambda i: (i, 0)),
            pl.BlockSpec((D, 64), lambda i: (0, 0)),
            pl.BlockSpec((NC, 256, 16), lambda i: (0, i, 0)),
        ],
        out_specs=pl.BlockSpec((256, 64), lambda i: (i, 0)),
        out_shape=jax.ShapeDtypeStruct((NPAD, 64), jnp.float32),
    )(x_pad, W1, deg_part)


def _tc2_body(aggb, hsb, degb, b1, w2, ob):
    dinv = _dinv_block(degb[...])
    h1 = jnp.maximum((aggb[0] + aggb[1] + hsb[...]) * dinv + b1[...], 0.0)
    ob[...] = jnp.dot(h1, w2[...], preferred_element_type=jnp.float32) * dinv


def _tc_mid(agg1, hs1, deg_part, b1r, W2):
    return pl.pallas_call(
        _tc2_body,
        grid=(NPAD // 256,),
        in_specs=[
            pl.BlockSpec((NC, 256, 64), lambda i: (0, i, 0)),
            pl.BlockSpec((256, 64), lambda i: (i, 0)),
            pl.BlockSpec((NC, 256, 16), lambda i: (0, i, 0)),
            pl.BlockSpec((1, 64), lambda i: (0, 0)),
            pl.BlockSpec((64, 64), lambda i: (0, 0)),
        ],
        out_specs=pl.BlockSpec((256, 64), lambda i: (i, 0)),
        out_shape=jax.ShapeDtypeStruct((NPAD, 64), jnp.float32),
    )(agg1, hs1, deg_part, b1r, W2)


def _tc3_body(aggb, hsb, degb, b2, ob):
    dinv = _dinv_block(degb[...])
    ob[...] = jnp.maximum((aggb[0] + aggb[1] + hsb[...]) * dinv + b2[...], 0.0)


def _tc_post(agg2, hs2, deg_part, b2r):
    return pl.pallas_call(
        _tc3_body,
        grid=(NPAD // 256,),
        in_specs=[
            pl.BlockSpec((NC, 256, 64), lambda i: (0, i, 0)),
            pl.BlockSpec((256, 64), lambda i: (i, 0)),
            pl.BlockSpec((NC, 256, 16), lambda i: (0, i, 0)),
            pl.BlockSpec((1, 64), lambda i: (0, 0)),
        ],
        out_specs=pl.BlockSpec((256, 64), lambda i: (i, 0)),
        out_shape=jax.ShapeDtypeStruct((NPAD, 64), jnp.float32),
    )(agg2, hs2, deg_part, b2r)


def _tc4_body(sumb, cntb, wf1, bf1, wf2, bf2, ob):
    cnt = cntb[0, :, 0] + cntb[1, :, 0]
    emb = (sumb[0] + sumb[1]) / jnp.maximum(cnt, 1.0)[:, None]
    h = jnp.maximum(
        jnp.dot(emb, wf1[...], preferred_element_type=jnp.float32) + bf1[...],
        0.0)
    z = jnp.dot(h, wf2[...], preferred_element_type=jnp.float32) + bf2[...]
    ob[...] = 1.0 / (1.0 + jnp.exp(-z))


def _tc_head(sums, cnts, Wf1p, bf1p, Wf2p, bf2p):
    return pl.pallas_call(
        _tc4_body,
        grid=(1,),
        in_specs=[
            pl.BlockSpec((NC, G, 64), lambda i: (0, 0, 0)),
            pl.BlockSpec((NC, G, 16), lambda i: (0, 0, 0)),
            pl.BlockSpec((64, 128), lambda i: (0, 0)),
            pl.BlockSpec((1, 128), lambda i: (0, 0)),
            pl.BlockSpec((128, 128), lambda i: (0, 0)),
            pl.BlockSpec((1, 128), lambda i: (0, 0)),
        ],
        out_specs=pl.BlockSpec((G, 128), lambda i: (0, 0)),
        out_shape=jax.ShapeDtypeStruct((G, 128), jnp.float32),
    )(sums, cnts, Wf1p, bf1p, Wf2p, bf2p)


def kernel(x, edge_index, batch, W1, b1, W2, b2, Wf1, bf1, Wf2, bf2):
    i32 = jnp.int32
    src2 = edge_index[0].astype(i32).reshape(NW, EPT)
    dst2 = edge_index[1].astype(i32).reshape(NW, EPT)
    pad = jnp.full((NW, ECH * 128 - EPT), TRASH_N, dtype=i32)
    src_idx = jnp.concatenate([src2, pad], axis=1).reshape(NW, ECH, 128)
    dst_idx = jnp.concatenate([dst2, pad], axis=1).reshape(NW, ECH, 128)

    npw = NPAD // NW
    j = jnp.arange(PCH * 128, dtype=i32)[None, :]
    pos = jnp.arange(NW, dtype=i32)[:, None] * npw + j
    valid = (j < npw) & (pos < N)
    pool_src = jnp.where(valid, pos, TRASH_N).reshape(NW, PCH, 128)
    batch_pad = jnp.concatenate(
        [batch.astype(i32), jnp.full((NPAD - N,), TRASH_G, dtype=i32)])
    pool_dst = jnp.where(valid, batch_pad[jnp.minimum(pos, NPAD - 1)],
                         TRASH_G).reshape(NW, PCH, 128)

    z16 = jnp.zeros((NPAD, 16), jnp.float32)
    z64 = jnp.zeros((NPAD, 64), jnp.float32)
    ones16 = jnp.ones((128, 16), jnp.float32)
    x_pad = jnp.concatenate(
        [x, jnp.zeros((NPAD - N, D), jnp.float32)], axis=0)

    deg_part, cnt_part = _sc_counts(dst_idx, pool_dst, ones16, z16)

    hs1 = _tc_pre(x_pad, W1, deg_part)
    agg1 = _sc_gather_scatter(ECH, NPAD, src_idx, dst_idx, hs1, z64)
    hs2 = _tc_mid(agg1, hs1, deg_part, b1.reshape(1, 64), W2)
    agg2 = _sc_gather_scatter(ECH, NPAD, src_idx, dst_idx, hs2, z64)
    h2 = _tc_post(agg2, hs2, deg_part, b2.reshape(1, 64))
    sums = _sc_gather_scatter(PCH, GPAD, pool_src, pool_dst, h2, z64)

    Wf1p = jnp.zeros((64, 128), jnp.float32).at[:, :32].set(Wf1)
    bf1p = jnp.zeros((1, 128), jnp.float32).at[0, :32].set(bf1)
    Wf2p = jnp.zeros((128, 128), jnp.float32).at[:32, :1].set(Wf2)
    bf2p = jnp.zeros((1, 128), jnp.float32).at[0, 0].set(bf2[0])
    out = _tc_head(sums, cnt_part, Wf1p, bf1p, Wf2p, bf2p)
    return out[:, 0]


# speedup vs baseline: 16.5832x; 16.5832x over previous
import functools

import jax
import jax.numpy as jnp
from jax import lax
from jax.experimental import pallas as pl
from jax.experimental.pallas import tpu as pltpu
from jax.experimental.pallas import tpu_sc as plsc

N = 10000
E = 320000
D = 128
G = 256

NC = 2
NS = 16
NW = NC * NS

NPAD = 10240
GPAD = 384
TRASH_N = NPAD - 1
TRASH_G = GPAD - 1

EPT = E // NW
ECH = 79
PCH = 3

NPT = NPAD // NS
GPT = GPAD // NS


def _tile_id():
    return lax.axis_index("c") * NS + lax.axis_index("s")


def _counts_body(dst_idx, pool_idx, ones16, z16,
                 deg_out, cnt_out,
                 idx_v, pidx_v, ones_v, deg_sh, cnt_sh):
    c = lax.axis_index("c")
    s = lax.axis_index("s")
    w = _tile_id()
    pltpu.sync_copy(z16.at[pl.ds(s * NPT, NPT)], deg_sh.at[pl.ds(s * NPT, NPT)])
    pltpu.sync_copy(z16.at[pl.ds(s * GPT, GPT)], cnt_sh.at[pl.ds(s * GPT, GPT)])
    pltpu.sync_copy(ones16, ones_v)
    pltpu.sync_copy(dst_idx.at[w], idx_v)
    pltpu.sync_copy(pool_idx.at[w], pidx_v)
    plsc.subcore_barrier()

    def chunk(i, carry):
        pltpu.sync_copy(ones_v, deg_sh.at[idx_v.at[i]], add=True)
        return carry

    lax.fori_loop(0, ECH, chunk, 0)
    for j in range(PCH):
        pltpu.sync_copy(ones_v, cnt_sh.at[pidx_v.at[j]], add=True)
    plsc.subcore_barrier()
    pltpu.sync_copy(deg_sh.at[pl.ds(s * NPT, NPT)],
                    deg_out.at[c, pl.ds(s * NPT, NPT)])
    pltpu.sync_copy(cnt_sh.at[pl.ds(s * GPT, GPT)],
                    cnt_out.at[c, pl.ds(s * GPT, GPT)])


def _sc_counts(dst_idx, pool_idx, ones16, z16):
    mesh = plsc.VectorSubcoreMesh(core_axis_name="c", subcore_axis_name="s")
    fn = pl.kernel(
        _counts_body,
        out_type=(jax.ShapeDtypeStruct((NC, NPAD, 16), jnp.float32),
                  jax.ShapeDtypeStruct((NC, GPAD, 16), jnp.float32)),
        mesh=mesh,
        scratch_types=[
            pltpu.VMEM((ECH, 128), jnp.int32),
            pltpu.VMEM((PCH, 128), jnp.int32),
            pltpu.VMEM((128, 16), jnp.float32),
            pltpu.VMEM_SHARED((NPAD, 16), jnp.float32),
            pltpu.VMEM_SHARED((GPAD, 16), jnp.float32),
        ],
        compiler_params=pltpu.CompilerParams(use_tc_tiling_on_sc=False),
    )
    return fn(dst_idx, pool_idx, ones16, z16)


def _gs_body(nch, rows_pt, src_idx, dst_idx, table, zrows, agg_out,
             sidx_v, didx_v, rows_v, acc_sh, sem):
    c = lax.axis_index("c")
    s = lax.axis_index("s")
    w = _tile_id()
    pltpu.sync_copy(zrows.at[pl.ds(s * rows_pt, rows_pt)],
                    acc_sh.at[pl.ds(s * rows_pt, rows_pt)])
    pltpu.sync_copy(src_idx.at[w], sidx_v)
    pltpu.sync_copy(dst_idx.at[w], didx_v)
    plsc.subcore_barrier()

    def chunk(i, carry):
        pltpu.async_copy(table.at[sidx_v.at[i]], rows_v, sem).wait()
        pltpu.sync_copy(rows_v, acc_sh.at[didx_v.at[i]], add=True)
        return carry

    lax.fori_loop(0, nch, chunk, 0)
    plsc.subcore_barrier()
    pltpu.sync_copy(acc_sh.at[pl.ds(s * rows_pt, rows_pt)],
                    agg_out.at[c, pl.ds(s * rows_pt, rows_pt)])


def _sc_gather_scatter(nch, acc_rows, src_idx, dst_idx, table, zrows):
    rows_pt = acc_rows // NS
    mesh = plsc.VectorSubcoreMesh(core_axis_name="c", subcore_axis_name="s")
    fn = pl.kernel(
        functools.partial(_gs_body, nch, rows_pt),
        out_type=jax.ShapeDtypeStruct((NC, acc_rows, 64), jnp.float32),
        mesh=mesh,
        scratch_types=[
            pltpu.VMEM((nch, 128), jnp.int32),
            pltpu.VMEM((nch, 128), jnp.int32),
            pltpu.VMEM((128, 64), jnp.float32),
            pltpu.VMEM_SHARED((acc_rows, 64), jnp.float32),
            pltpu.SemaphoreType.DMA,
        ],
        compiler_params=pltpu.CompilerParams(use_tc_tiling_on_sc=False),
    )
    return fn(src_idx, dst_idx, table, zrows)


def _dinv_block(degb):
    deg = degb[0, :, 0] + degb[1, :, 0] + 1.0
    return lax.rsqrt(deg)[:, None]


def _tc1_body(xb, w1, degb, ob):
    h = jnp.dot(xb[...], w1[...], preferred_element_type=jnp.float32)
    ob[...] = h * _dinv_block(degb[...])


def _tc_pre(x_pad, W1, deg_part):
    return pl.pallas_call(
        _tc1_body,
        grid=(NPAD // 256,),
        in_specs=[
            pl.BlockSpec((256, D), lambda i: (i, 0)),
            pl.BlockSpec((D, 64), lambda i: (0, 0)),
            pl.BlockSpec((NC, 256, 16), lambda i: (0, i, 0)),
        ],
        out_specs=pl.BlockSpec((256, 64), lambda i: (i, 0)),
        out_shape=jax.ShapeDtypeStruct((NPAD, 64), jnp.float32),
    )(x_pad, W1, deg_part)


def _tc2_body(aggb, hsb, degb, b1, w2, ob):
    dinv = _dinv_block(degb[...])
    h1 = jnp.maximum((aggb[0] + aggb[1] + hsb[...]) * dinv + b1[...], 0.0)
    ob[...] = jnp.dot(h1, w2[...], preferred_element_type=jnp.float32) * dinv


def _tc_mid(agg1, hs1, deg_part, b1r, W2):
    return pl.pallas_call(
        _tc2_body,
        grid=(NPAD // 256,),
        in_specs=[
            pl.BlockSpec((NC, 256, 64), lambda i: (0, i, 0)),
            pl.BlockSpec((256, 64), lambda i: (i, 0)),
            pl.BlockSpec((NC, 256, 16), lambda i: (0, i, 0)),
            pl.BlockSpec((1, 64), lambda i: (0, 0)),
            pl.BlockSpec((64, 64), lambda i: (0, 0)),
        ],
        out_specs=pl.BlockSpec((256, 64), lambda i: (i, 0)),
        out_shape=jax.ShapeDtypeStruct((NPAD, 64), jnp.float32),
    )(agg1, hs1, deg_part, b1r, W2)


def _tc3_body(aggb, hsb, degb, b2, ob):
    dinv = _dinv_block(degb[...])
    ob[...] = jnp.maximum((aggb[0] + aggb[1] + hsb[...]) * dinv + b2[...], 0.0)


def _tc_post(agg2, hs2, deg_part, b2r):
    return pl.pallas_call(
        _tc3_body,
        grid=(NPAD // 256,),
        in_specs=[
            pl.BlockSpec((NC, 256, 64), lambda i: (0, i, 0)),
            pl.BlockSpec((256, 64), lambda i: (i, 0)),
            pl.BlockSpec((NC, 256, 16), lambda i: (0, i, 0)),
            pl.BlockSpec((1, 64), lambda i: (0, 0)),
        ],
        out_specs=pl.BlockSpec((256, 64), lambda i: (i, 0)),
        out_shape=jax.ShapeDtypeStruct((NPAD, 64), jnp.float32),
    )(agg2, hs2, deg_part, b2r)


def _tc4_body(sumb, cntb, wf1, bf1, wf2, bf2, ob):
    cnt = cntb[0, :, 0] + cntb[1, :, 0]
    emb = (sumb[0] + sumb[1]) / jnp.maximum(cnt, 1.0)[:, None]
    h = jnp.maximum(
        jnp.dot(emb, wf1[...], preferred_element_type=jnp.float32) + bf1[...],
        0.0)
    z = jnp.dot(h, wf2[...], preferred_element_type=jnp.float32) + bf2[...]
    ob[...] = 1.0 / (1.0 + jnp.exp(-z))


def _tc_head(sums, cnts, Wf1p, bf1p, Wf2p, bf2p):
    return pl.pallas_call(
        _tc4_body,
        grid=(1,),
        in_specs=[
            pl.BlockSpec((NC, G, 64), lambda i: (0, 0, 0)),
            pl.BlockSpec((NC, G, 16), lambda i: (0, 0, 0)),
            pl.BlockSpec((64, 128), lambda i: (0, 0)),
            pl.BlockSpec((1, 128), lambda i: (0, 0)),
            pl.BlockSpec((128, 128), lambda i: (0, 0)),
            pl.BlockSpec((1, 128), lambda i: (0, 0)),
        ],
        out_specs=pl.BlockSpec((G, 128), lambda i: (0, 0)),
        out_shape=jax.ShapeDtypeStruct((G, 128), jnp.float32),
    )(sums, cnts, Wf1p, bf1p, Wf2p, bf2p)


def kernel(x, edge_index, batch, W1, b1, W2, b2, Wf1, bf1, Wf2, bf2):
    i32 = jnp.int32
    src2 = edge_index[0].astype(i32).reshape(NW, EPT)
    dst2 = edge_index[1].astype(i32).reshape(NW, EPT)
    pad = jnp.full((NW, ECH * 128 - EPT), TRASH_N, dtype=i32)
    src_idx = jnp.concatenate([src2, pad], axis=1).reshape(NW, ECH, 128)
    dst_idx = jnp.concatenate([dst2, pad], axis=1).reshape(NW, ECH, 128)

    npw = NPAD // NW
    j = jnp.arange(PCH * 128, dtype=i32)[None, :]
    pos = jnp.arange(NW, dtype=i32)[:, None] * npw + j
    valid = (j < npw) & (pos < N)
    pool_src = jnp.where(valid, pos, TRASH_N).reshape(NW, PCH, 128)
    batch_pad = jnp.concatenate(
        [batch.astype(i32), jnp.full((NPAD - N,), TRASH_G, dtype=i32)])
    pool_dst = jnp.where(valid, batch_pad[jnp.minimum(pos, NPAD - 1)],
                         TRASH_G).reshape(NW, PCH, 128)

    z16 = jnp.zeros((NPAD, 16), jnp.float32)
    z64 = jnp.zeros((NPAD, 64), jnp.float32)
    ones16 = jnp.ones((128, 16), jnp.float32)
    x_pad = jnp.concatenate(
        [x, jnp.zeros((NPAD - N, D), jnp.float32)], axis=0)

    deg_part, cnt_part = _sc_counts(dst_idx, pool_dst, ones16, z16)

    hs1 = _tc_pre(x_pad, W1, deg_part)
    agg1 = _sc_gather_scatter(ECH, NPAD, src_idx, dst_idx, hs1, z64)
    hs2 = _tc_mid(agg1, hs1, deg_part, b1.reshape(1, 64), W2)
    agg2 = _sc_gather_scatter(ECH, NPAD, src_idx, dst_idx, hs2, z64)
    h2 = _tc_post(agg2, hs2, deg_part, b2.reshape(1, 64))
    sums = _sc_gather_scatter(PCH, GPAD, pool_src, pool_dst, h2, z64)

    Wf1p = jnp.zeros((64, 128), jnp.float32).at[:, :32].set(Wf1)
    bf1p = jnp.zeros((1, 128), jnp.float32).at[0, :32].set(bf1)
    Wf2p = jnp.zeros((128, 128), jnp.float32).at[:32, :1].set(Wf2)
    bf2p = jnp.zeros((1, 128), jnp.float32).at[0, 0].set(bf2[0])
    out = _tc_head(sums, cnt_part, Wf1p, bf1p, Wf2p, bf2p)
    return out[:, 0]
